# baseline (speedup 1.0000x reference)
import jax
import jax.numpy as jnp
from jax.experimental import pallas as pl

F_BLOCK = 512


def _decode_matmul_kernel(x_ref, binary_ref, scale_ref, bias_ref, out_ref):
    byte = binary_ref[...]
    nbits, G, F = byte.shape
    k = jax.lax.broadcasted_iota(jnp.int32, (1, 1, 8, 1), 2)
    bits = (byte[:, :, None, :] >> (7 - k)) & 1
    bits = bits.reshape(nbits, G * 8, F).astype(jnp.float32)
    s = scale_ref[...]
    w = 2.0 * jnp.sum(bits * s[:, None, :], axis=0) - jnp.sum(s, axis=0)[None, :]
    out_ref[...] = (
        jnp.dot(x_ref[...], w, preferred_element_type=jnp.float32,
                precision=jax.lax.Precision.DEFAULT)
        + bias_ref[...]
    )


def kernel(x, binary, scale, bias):
    size_out = x.shape[:-1] + (bias.shape[-1],)
    x2 = x.reshape(-1, x.shape[-1])
    T, nx = x2.shape
    nbits = scale.shape[1]
    nf = scale.shape[2]
    G = nx // 8

    binary3 = binary[0, :nbits].astype(jnp.int32)
    scale2 = scale[0]
    bias2 = bias.reshape(1, nf)

    out = pl.pallas_call(
        _decode_matmul_kernel,
        grid=(nf // F_BLOCK,),
        in_specs=[
            pl.BlockSpec((T, nx), lambda j: (0, 0)),
            pl.BlockSpec((nbits, G, F_BLOCK), lambda j: (0, 0, j)),
            pl.BlockSpec((nbits, F_BLOCK), lambda j: (0, j)),
            pl.BlockSpec((1, F_BLOCK), lambda j: (0, j)),
        ],
        out_specs=pl.BlockSpec((T, F_BLOCK), lambda j: (0, j)),
        out_shape=jax.ShapeDtypeStruct((T, nf), jnp.float32),
    )(x2, binary3, scale2, bias2)
    return out.reshape(size_out)


# speedup vs baseline: 1.1981x; 1.1981x over previous
import jax
import jax.numpy as jnp
from jax.experimental import pallas as pl

F_BLOCK = 512


def _decode_matmul_kernel(x_ref, binary_ref, scale_ref, bias_ref, out_ref):
    byte = binary_ref[...]
    nbits, G, F = byte.shape
    packed = byte[0]
    for b in range(1, nbits):
        packed = packed | (byte[b] << (8 * b))
    pk = packed[:, None, :]
    k = jax.lax.broadcasted_iota(jnp.int32, (1, 8, 1), 1)
    msb = jnp.int32(-(2**31))
    neg_s = jax.lax.bitcast_convert_type(-scale_ref[...], jnp.int32)
    w = None
    for b in range(nbits):
        sgn = (pk << (24 - 8 * b + k)) & msb
        wb = jax.lax.bitcast_convert_type(neg_s[b][None, None, :] ^ sgn,
                                          jnp.float32)
        w = wb if w is None else w + wb
    w = w.reshape(G * 8, F)
    out_ref[...] = (
        jnp.dot(x_ref[...], w, preferred_element_type=jnp.float32,
                precision=jax.lax.Precision.DEFAULT)
        + bias_ref[...]
    )


def kernel(x, binary, scale, bias):
    size_out = x.shape[:-1] + (bias.shape[-1],)
    x2 = x.reshape(-1, x.shape[-1])
    T, nx = x2.shape
    nbits = scale.shape[1]
    nf = scale.shape[2]
    G = nx // 8

    binary3 = binary[0, :nbits].astype(jnp.int32)
    scale2 = scale[0]
    bias2 = bias.reshape(1, nf)

    out = pl.pallas_call(
        _decode_matmul_kernel,
        grid=(nf // F_BLOCK,),
        in_specs=[
            pl.BlockSpec((T, nx), lambda j: (0, 0)),
            pl.BlockSpec((nbits, G, F_BLOCK), lambda j: (0, 0, j)),
            pl.BlockSpec((nbits, F_BLOCK), lambda j: (0, j)),
            pl.BlockSpec((1, F_BLOCK), lambda j: (0, j)),
        ],
        out_specs=pl.BlockSpec((T, F_BLOCK), lambda j: (0, j)),
        out_shape=jax.ShapeDtypeStruct((T, nf), jnp.float32),
    )(x2, binary3, scale2, bias2)
    return out.reshape(size_out)
